# baseline (speedup 1.0000x reference)
import functools

import jax
import jax.numpy as jnp
from jax.experimental import pallas as pl
from jax.experimental.pallas import tpu as pltpu

EMB_DIM = 128
_VMEM_PARAMS = pltpu.CompilerParams(vmem_limit_bytes=110 * 1024 * 1024)


def _intra_block(adj, mat, emb):
    r = jnp.sum(emb, axis=1)
    logits = mat * r[None, :]
    m = jnp.max(logits, axis=1, keepdims=True)
    e = jnp.exp(logits - m)
    s = e / jnp.sum(e, axis=1, keepdims=True)
    w = s * adj
    w = w / (jnp.sum(w, axis=1, keepdims=True) + 1e-8)
    return jnp.dot(w, emb, preferred_element_type=jnp.float32)


def _gate(e, W, b):
    return jnp.exp(jnp.sum(e * W, axis=1, keepdims=True) + b)


def _inter(W, b, e0, e1, e2, e3):
    g0 = _gate(e0, W, b)
    g1 = _gate(e1, W, b)
    g2 = _gate(e2, W, b)
    g3 = _gate(e3, W, b)
    s = g0 + g1 + g2 + g3
    return (g0 / s) * e0 + (g1 / s) * e1 + (g2 / s) * e2 + (g3 / s) * e3


def _item_kernel(adj0_ref, adj1_ref, adj2_ref, adj3_ref, adj4_ref, avp_ref,
                 avcb_ref, avcm_ref, item_ref, pri_ref, cb_ref, cm_ref,
                 mvp_ref, mvcb_ref, mvcm_ref, W_ref, b_ref, out_ref, *, br):
    i = pl.program_id(0)
    e0 = item_ref[pl.ds(i * br, br), :]
    e1 = _intra_block(avp_ref[...], mvp_ref[...], pri_ref[...])
    e2 = _intra_block(avcb_ref[...], mvcb_ref[...], cb_ref[...])
    e3 = _intra_block(avcm_ref[...], mvcm_ref[...], cm_ref[...])
    gated = _inter(W_ref[...], b_ref[...], e0, e1, e2, e3)
    itb = item_ref[...].astype(jnp.bfloat16)
    big = jnp.concatenate([
        jnp.dot(a[...].astype(jnp.bfloat16), itb,
                preferred_element_type=jnp.float32)
        for a in (adj0_ref, adj1_ref, adj2_ref, adj3_ref, adj4_ref)
    ], axis=0)
    out_ref[...] = gated + big


def _item_update(adjacency, a_vp, a_vcb, a_vcm, item, pri, cb, cm,
                 m_vp, m_vcb, m_vcm, W, b):
    n = item.shape[0]
    br = 400 if n % 400 == 0 else n
    hr = br // 5
    grid = (n // br,)
    return pl.pallas_call(
        functools.partial(_item_kernel, br=br),
        grid=grid,
        in_specs=[
            pl.BlockSpec((hr, n), lambda i: (5 * i, 0)),
            pl.BlockSpec((hr, n), lambda i: (5 * i + 1, 0)),
            pl.BlockSpec((hr, n), lambda i: (5 * i + 2, 0)),
            pl.BlockSpec((hr, n), lambda i: (5 * i + 3, 0)),
            pl.BlockSpec((hr, n), lambda i: (5 * i + 4, 0)),
            pl.BlockSpec((br, a_vp.shape[1]), lambda i: (i, 0)),
            pl.BlockSpec((br, a_vcb.shape[1]), lambda i: (i, 0)),
            pl.BlockSpec((br, a_vcm.shape[1]), lambda i: (i, 0)),
            pl.BlockSpec((n, EMB_DIM), lambda i: (0, 0)),
            pl.BlockSpec(pri.shape, lambda i: (0, 0)),
            pl.BlockSpec(cb.shape, lambda i: (0, 0)),
            pl.BlockSpec(cm.shape, lambda i: (0, 0)),
            pl.BlockSpec((br, 1), lambda i: (i, 0)),
            pl.BlockSpec((br, 1), lambda i: (i, 0)),
            pl.BlockSpec((br, 1), lambda i: (i, 0)),
            pl.BlockSpec((1, EMB_DIM), lambda i: (0, 0)),
            pl.BlockSpec((1, 1), lambda i: (0, 0)),
        ],
        out_specs=pl.BlockSpec((br, EMB_DIM), lambda i: (i, 0)),
        out_shape=jax.ShapeDtypeStruct((n, EMB_DIM), jnp.float32),
        compiler_params=_VMEM_PARAMS,
    )(adjacency, adjacency, adjacency, adjacency, adjacency, a_vp, a_vcb,
      a_vcm, item, pri, cb, cm, m_vp, m_vcb, m_vcm, W, b)


def _chunks(n, w=2048):
    out = []
    off = 0
    while off < n:
        out.append((off, min(w, n - off)))
        off += w
    return out


def _stream_intra(adj_ref, mat, item_ref, n):
    num = None
    for off, w in _chunks(n):
        itb = item_ref[pl.ds(off, w), :]
        r = jnp.sum(itb, axis=1)
        e = jnp.exp(mat * r[None, :])
        wgt = e * adj_ref[:, pl.ds(off, w)]
        pnum = jnp.dot(wgt, itb, preferred_element_type=jnp.float32)
        ps = jnp.sum(wgt, axis=1, keepdims=True)
        pd = jnp.sum(e, axis=1, keepdims=True)
        if num is None:
            num, s, d = pnum, ps, pd
        else:
            num, s, d = num + pnum, s + ps, d + pd
    return num / (s + 1e-8 * d)


def _pcc_kernel(apv_ref, acbv_ref, acmv_ref, item_ref,
                apcb_ref, apcm_ref, acbp_ref, acbcm_ref, acmp_ref, acmcb_ref,
                pri_ref, cb_ref, cm_ref,
                mpv_ref, mcbv_ref, mcmv_ref, mpcb_ref, mpcm_ref, mcbp_ref,
                mcbcm_ref, mcmp_ref, mcmcb_ref,
                Wp_ref, bp_ref, Wcb_ref, bcb_ref, Wcm_ref, bcm_ref,
                pr_out, cbn_out, cmn_out, *, n):
    p = pri_ref[...]
    c_b = cb_ref[...]
    c_m = cm_ref[...]
    pr_out[...] = _inter(
        Wp_ref[...], bp_ref[...], p,
        _stream_intra(apv_ref, mpv_ref[...], item_ref, n),
        _intra_block(apcb_ref[...], mpcb_ref[...], c_b),
        _intra_block(apcm_ref[...], mpcm_ref[...], c_m))
    cbn_out[...] = _inter(
        Wcb_ref[...], bcb_ref[...], c_b,
        _intra_block(acbp_ref[...], mcbp_ref[...], p),
        _stream_intra(acbv_ref, mcbv_ref[...], item_ref, n),
        _intra_block(acbcm_ref[...], mcbcm_ref[...], c_m))
    cmn_out[...] = _inter(
        Wcm_ref[...], bcm_ref[...], c_m,
        _intra_block(acmp_ref[...], mcmp_ref[...], p),
        _stream_intra(acmv_ref, mcmv_ref[...], item_ref, n),
        _intra_block(acmcb_ref[...], mcmcb_ref[...], c_b))


def _pcc_update(a_pv, a_pcb, a_pcm, a_cbp, a_cbv, a_cbcm, a_cmp, a_cmv,
                a_cmcb, item, pri, cb, cm, m_pv, m_pcb, m_pcm, m_cbp, m_cbv,
                m_cbcm, m_cmp, m_cmv, m_cmcb, Wp, bp, Wcb, bcb, Wcm, bcm):
    n = item.shape[0]
    np_, ncb, ncm = pri.shape[0], cb.shape[0], cm.shape[0]
    return pl.pallas_call(
        functools.partial(_pcc_kernel, n=n),
        out_shape=(
            jax.ShapeDtypeStruct((np_, EMB_DIM), jnp.float32),
            jax.ShapeDtypeStruct((ncb, EMB_DIM), jnp.float32),
            jax.ShapeDtypeStruct((ncm, EMB_DIM), jnp.float32),
        ),
        compiler_params=_VMEM_PARAMS,
    )(a_pv, a_cbv, a_cmv, item, a_pcb, a_pcm, a_cbp, a_cbcm, a_cmp, a_cmcb,
      pri, cb, cm, m_pv, m_cbv, m_cmv, m_pcb, m_pcm, m_cbp, m_cbcm,
      m_cmp, m_cmcb, Wp, bp, Wcb, bcb, Wcm, bcm)


def _pr_kernel(apv_ref, item_ref, apcb_ref, apcm_ref, pri_ref, cb_ref,
               cm_ref, mpv_ref, mpcb_ref, mpcm_ref, Wp_ref, bp_ref, pr_out,
               *, n):
    pr_out[...] = _inter(
        Wp_ref[...], bp_ref[...], pri_ref[...],
        _stream_intra(apv_ref, mpv_ref[...], item_ref, n),
        _intra_block(apcb_ref[...], mpcb_ref[...], cb_ref[...]),
        _intra_block(apcm_ref[...], mpcm_ref[...], cm_ref[...]))


def _pr_update(a_pv, a_pcb, a_pcm, item, pri, cb, cm, m_pv, m_pcb, m_pcm,
               Wp, bp):
    n = item.shape[0]
    np_ = pri.shape[0]
    return pl.pallas_call(
        functools.partial(_pr_kernel, n=n),
        out_shape=jax.ShapeDtypeStruct((np_, EMB_DIM), jnp.float32),
        compiler_params=_VMEM_PARAMS,
    )(a_pv, item, a_pcb, a_pcm, pri, cb, cm, m_pv, m_pcb, m_pcm, Wp, bp)


def kernel(adjacency, adjacency_pv, adjacency_vp, adjacency_pcb,
           adjacency_cbp, adjacency_cbv, adjacency_vcb, adjacency_pcm,
           adjacency_cmp, adjacency_cmv, adjacency_vcm, adjacency_cbcm,
           adjacency_cmcb, item_emb, pri_emb, cateBig_emb, cateMiddle_emb,
           mat_vp, mat_vcb, mat_vcm, mat_pv, mat_pcb, mat_pcm, mat_cbp,
           mat_cbv, mat_cbcm, mat_cmp, mat_cmv, mat_cmcb, W_gi, b_gi,
           W_gp, b_gp, W_gcb, b_gcb, W_gcm, b_gcm):
    b_gi2 = b_gi.reshape(1, 1)
    b_gp2 = b_gp.reshape(1, 1)
    b_gcb2 = b_gcb.reshape(1, 1)
    b_gcm2 = b_gcm.reshape(1, 1)

    it1 = _item_update(adjacency, adjacency_vp, adjacency_vcb, adjacency_vcm,
                       item_emb, pri_emb, cateBig_emb, cateMiddle_emb,
                       mat_vp, mat_vcb, mat_vcm, W_gi, b_gi2)
    pr1, cb1, cm1 = _pcc_update(
        adjacency_pv, adjacency_pcb, adjacency_pcm, adjacency_cbp,
        adjacency_cbv, adjacency_cbcm, adjacency_cmp, adjacency_cmv,
        adjacency_cmcb, item_emb, pri_emb, cateBig_emb, cateMiddle_emb,
        mat_pv, mat_pcb, mat_pcm, mat_cbp, mat_cbv, mat_cbcm, mat_cmp,
        mat_cmv, mat_cmcb, W_gp, b_gp2, W_gcb, b_gcb2, W_gcm, b_gcm2)

    it2 = _item_update(adjacency, adjacency_vp, adjacency_vcb, adjacency_vcm,
                       it1, pr1, cb1, cm1, mat_vp, mat_vcb, mat_vcm,
                       W_gi, b_gi2)
    pr2 = _pr_update(adjacency_pv, adjacency_pcb, adjacency_pcm,
                     it1, pr1, cb1, cm1, mat_pv, mat_pcb, mat_pcm,
                     W_gp, b_gp2)
    return (it2, pr2)


# speedup vs baseline: 1.0153x; 1.0153x over previous
import functools

import jax
import jax.numpy as jnp
from jax.experimental import pallas as pl
from jax.experimental.pallas import tpu as pltpu

EMB_DIM = 128
_VMEM_PARAMS = pltpu.CompilerParams(vmem_limit_bytes=110 * 1024 * 1024)


def _intra_block(adj, mat, emb):
    r = jnp.sum(emb, axis=1)
    logits = mat * r[None, :]
    m = jnp.max(logits, axis=1, keepdims=True)
    e = jnp.exp(logits - m)
    s = e / jnp.sum(e, axis=1, keepdims=True)
    w = s * adj
    w = w / (jnp.sum(w, axis=1, keepdims=True) + 1e-8)
    return jnp.dot(w, emb, preferred_element_type=jnp.float32)


def _gate(e, W, b):
    return jnp.exp(jnp.sum(e * W, axis=1, keepdims=True) + b)


def _inter(W, b, e0, e1, e2, e3):
    g0 = _gate(e0, W, b)
    g1 = _gate(e1, W, b)
    g2 = _gate(e2, W, b)
    g3 = _gate(e3, W, b)
    s = g0 + g1 + g2 + g3
    return (g0 / s) * e0 + (g1 / s) * e1 + (g2 / s) * e2 + (g3 / s) * e3


def _item_kernel(adj0_ref, adj1_ref, avp_ref, avcb_ref, avcm_ref, item_ref,
                 pri_ref, cb_ref, cm_ref, mvp_ref, mvcb_ref, mvcm_ref,
                 W_ref, b_ref, out_ref, *, br):
    i = pl.program_id(0)
    e0 = item_ref[pl.ds(i * br, br), :]
    e1 = _intra_block(avp_ref[...], mvp_ref[...], pri_ref[...])
    e2 = _intra_block(avcb_ref[...], mvcb_ref[...], cb_ref[...])
    e3 = _intra_block(avcm_ref[...], mvcm_ref[...], cm_ref[...])
    gated = _inter(W_ref[...], b_ref[...], e0, e1, e2, e3)
    itb = item_ref[...].astype(jnp.bfloat16)
    big = jnp.concatenate([
        jnp.dot(a[...].astype(jnp.bfloat16), itb,
                preferred_element_type=jnp.float32)
        for a in (adj0_ref, adj1_ref)
    ], axis=0)
    out_ref[...] = gated + big


def _item_update(adjacency, a_vp, a_vcb, a_vcm, item, pri, cb, cm,
                 m_vp, m_vcb, m_vcm, W, b):
    n = item.shape[0]
    br = 400 if n % 400 == 0 else n
    hr = br // 2
    grid = (n // br,)
    return pl.pallas_call(
        functools.partial(_item_kernel, br=br),
        grid=grid,
        in_specs=[
            pl.BlockSpec((hr, n), lambda i: (2 * i, 0)),
            pl.BlockSpec((hr, n), lambda i: (2 * i + 1, 0)),
            pl.BlockSpec((br, a_vp.shape[1]), lambda i: (i, 0)),
            pl.BlockSpec((br, a_vcb.shape[1]), lambda i: (i, 0)),
            pl.BlockSpec((br, a_vcm.shape[1]), lambda i: (i, 0)),
            pl.BlockSpec((n, EMB_DIM), lambda i: (0, 0)),
            pl.BlockSpec(pri.shape, lambda i: (0, 0)),
            pl.BlockSpec(cb.shape, lambda i: (0, 0)),
            pl.BlockSpec(cm.shape, lambda i: (0, 0)),
            pl.BlockSpec((br, 1), lambda i: (i, 0)),
            pl.BlockSpec((br, 1), lambda i: (i, 0)),
            pl.BlockSpec((br, 1), lambda i: (i, 0)),
            pl.BlockSpec((1, EMB_DIM), lambda i: (0, 0)),
            pl.BlockSpec((1, 1), lambda i: (0, 0)),
        ],
        out_specs=pl.BlockSpec((br, EMB_DIM), lambda i: (i, 0)),
        out_shape=jax.ShapeDtypeStruct((n, EMB_DIM), jnp.float32),
        compiler_params=_VMEM_PARAMS,
    )(adjacency, adjacency, a_vp, a_vcb, a_vcm, item, pri, cb, cm,
      m_vp, m_vcb, m_vcm, W, b)


def _item_pr_kernel(adj0_ref, adj1_ref, avp_ref, avcb_ref, avcm_ref,
                    item_ref, pri_ref, cb_ref, cm_ref, mvp_ref, mvcb_ref,
                    mvcm_ref, W_ref, b_ref,
                    apv_ref, apcb_ref, apcm_ref, mpv_ref, mpcb_ref, mpcm_ref,
                    Wp_ref, bp_ref, out_ref, pr_out, *, br, nk, n):
    i = pl.program_id(0)
    e0 = item_ref[pl.ds(i * br, br), :]
    e1 = _intra_block(avp_ref[...], mvp_ref[...], pri_ref[...])
    e2 = _intra_block(avcb_ref[...], mvcb_ref[...], cb_ref[...])
    e3 = _intra_block(avcm_ref[...], mvcm_ref[...], cm_ref[...])
    gated = _inter(W_ref[...], b_ref[...], e0, e1, e2, e3)
    itb = item_ref[...].astype(jnp.bfloat16)
    big = jnp.concatenate([
        jnp.dot(a[...].astype(jnp.bfloat16), itb,
                preferred_element_type=jnp.float32)
        for a in (adj0_ref, adj1_ref)
    ], axis=0)
    out_ref[...] = gated + big

    @pl.when(i == nk - 1)
    def _pr():
        pr_out[...] = _inter(
            Wp_ref[...], bp_ref[...], pri_ref[...],
            _stream_intra(apv_ref, mpv_ref[...], item_ref, n),
            _intra_block(apcb_ref[...], mpcb_ref[...], cb_ref[...]),
            _intra_block(apcm_ref[...], mpcm_ref[...], cm_ref[...]))


def _item_pr_update(adjacency, a_vp, a_vcb, a_vcm, item, pri, cb, cm,
                    m_vp, m_vcb, m_vcm, W, b,
                    a_pv, a_pcb, a_pcm, m_pv, m_pcb, m_pcm, Wp, bp):
    n = item.shape[0]
    br = 400 if n % 400 == 0 else n
    hr = br // 2
    nk = n // br
    np_ = pri.shape[0]
    full = lambda x: pl.BlockSpec(x.shape, lambda i: (0, 0))
    return pl.pallas_call(
        functools.partial(_item_pr_kernel, br=br, nk=nk, n=n),
        grid=(nk,),
        in_specs=[
            pl.BlockSpec((hr, n), lambda i: (2 * i, 0)),
            pl.BlockSpec((hr, n), lambda i: (2 * i + 1, 0)),
            pl.BlockSpec((br, a_vp.shape[1]), lambda i: (i, 0)),
            pl.BlockSpec((br, a_vcb.shape[1]), lambda i: (i, 0)),
            pl.BlockSpec((br, a_vcm.shape[1]), lambda i: (i, 0)),
            pl.BlockSpec((n, EMB_DIM), lambda i: (0, 0)),
            full(pri), full(cb), full(cm),
            pl.BlockSpec((br, 1), lambda i: (i, 0)),
            pl.BlockSpec((br, 1), lambda i: (i, 0)),
            pl.BlockSpec((br, 1), lambda i: (i, 0)),
            full(W), pl.BlockSpec((1, 1), lambda i: (0, 0)),
            full(a_pv), full(a_pcb), full(a_pcm),
            full(m_pv), full(m_pcb), full(m_pcm),
            full(Wp), pl.BlockSpec((1, 1), lambda i: (0, 0)),
        ],
        out_specs=(
            pl.BlockSpec((br, EMB_DIM), lambda i: (i, 0)),
            pl.BlockSpec((np_, EMB_DIM), lambda i: (0, 0)),
        ),
        out_shape=(
            jax.ShapeDtypeStruct((n, EMB_DIM), jnp.float32),
            jax.ShapeDtypeStruct((np_, EMB_DIM), jnp.float32),
        ),
        compiler_params=_VMEM_PARAMS,
    )(adjacency, adjacency, a_vp, a_vcb, a_vcm, item, pri, cb, cm,
      m_vp, m_vcb, m_vcm, W, b, a_pv, a_pcb, a_pcm, m_pv, m_pcb, m_pcm,
      Wp, bp)


def _chunks(n, w=2048):
    out = []
    off = 0
    while off < n:
        out.append((off, min(w, n - off)))
        off += w
    return out


def _stream_intra(adj_ref, mat, item_ref, n):
    num = None
    for off, w in _chunks(n):
        itb = item_ref[pl.ds(off, w), :]
        r = jnp.sum(itb, axis=1)
        e = jnp.exp(mat * r[None, :])
        wgt = e * adj_ref[:, pl.ds(off, w)]
        pnum = jnp.dot(wgt, itb, preferred_element_type=jnp.float32)
        ps = jnp.sum(wgt, axis=1, keepdims=True)
        pd = jnp.sum(e, axis=1, keepdims=True)
        if num is None:
            num, s, d = pnum, ps, pd
        else:
            num, s, d = num + pnum, s + ps, d + pd
    return num / (s + 1e-8 * d)


def _pcc_kernel(apv_ref, acbv_ref, acmv_ref, item_ref,
                apcb_ref, apcm_ref, acbp_ref, acbcm_ref, acmp_ref, acmcb_ref,
                pri_ref, cb_ref, cm_ref,
                mpv_ref, mcbv_ref, mcmv_ref, mpcb_ref, mpcm_ref, mcbp_ref,
                mcbcm_ref, mcmp_ref, mcmcb_ref,
                Wp_ref, bp_ref, Wcb_ref, bcb_ref, Wcm_ref, bcm_ref,
                pr_out, cbn_out, cmn_out, *, n):
    p = pri_ref[...]
    c_b = cb_ref[...]
    c_m = cm_ref[...]
    pr_out[...] = _inter(
        Wp_ref[...], bp_ref[...], p,
        _stream_intra(apv_ref, mpv_ref[...], item_ref, n),
        _intra_block(apcb_ref[...], mpcb_ref[...], c_b),
        _intra_block(apcm_ref[...], mpcm_ref[...], c_m))
    cbn_out[...] = _inter(
        Wcb_ref[...], bcb_ref[...], c_b,
        _intra_block(acbp_ref[...], mcbp_ref[...], p),
        _stream_intra(acbv_ref, mcbv_ref[...], item_ref, n),
        _intra_block(acbcm_ref[...], mcbcm_ref[...], c_m))
    cmn_out[...] = _inter(
        Wcm_ref[...], bcm_ref[...], c_m,
        _intra_block(acmp_ref[...], mcmp_ref[...], p),
        _stream_intra(acmv_ref, mcmv_ref[...], item_ref, n),
        _intra_block(acmcb_ref[...], mcmcb_ref[...], c_b))


def _pcc_update(a_pv, a_pcb, a_pcm, a_cbp, a_cbv, a_cbcm, a_cmp, a_cmv,
                a_cmcb, item, pri, cb, cm, m_pv, m_pcb, m_pcm, m_cbp, m_cbv,
                m_cbcm, m_cmp, m_cmv, m_cmcb, Wp, bp, Wcb, bcb, Wcm, bcm):
    n = item.shape[0]
    np_, ncb, ncm = pri.shape[0], cb.shape[0], cm.shape[0]
    return pl.pallas_call(
        functools.partial(_pcc_kernel, n=n),
        out_shape=(
            jax.ShapeDtypeStruct((np_, EMB_DIM), jnp.float32),
            jax.ShapeDtypeStruct((ncb, EMB_DIM), jnp.float32),
            jax.ShapeDtypeStruct((ncm, EMB_DIM), jnp.float32),
        ),
        compiler_params=_VMEM_PARAMS,
    )(a_pv, a_cbv, a_cmv, item, a_pcb, a_pcm, a_cbp, a_cbcm, a_cmp, a_cmcb,
      pri, cb, cm, m_pv, m_cbv, m_cmv, m_pcb, m_pcm, m_cbp, m_cbcm,
      m_cmp, m_cmcb, Wp, bp, Wcb, bcb, Wcm, bcm)


def _pr_kernel(apv_ref, item_ref, apcb_ref, apcm_ref, pri_ref, cb_ref,
               cm_ref, mpv_ref, mpcb_ref, mpcm_ref, Wp_ref, bp_ref, pr_out,
               *, n):
    pr_out[...] = _inter(
        Wp_ref[...], bp_ref[...], pri_ref[...],
        _stream_intra(apv_ref, mpv_ref[...], item_ref, n),
        _intra_block(apcb_ref[...], mpcb_ref[...], cb_ref[...]),
        _intra_block(apcm_ref[...], mpcm_ref[...], cm_ref[...]))


def _pr_update(a_pv, a_pcb, a_pcm, item, pri, cb, cm, m_pv, m_pcb, m_pcm,
               Wp, bp):
    n = item.shape[0]
    np_ = pri.shape[0]
    return pl.pallas_call(
        functools.partial(_pr_kernel, n=n),
        out_shape=jax.ShapeDtypeStruct((np_, EMB_DIM), jnp.float32),
        compiler_params=_VMEM_PARAMS,
    )(a_pv, item, a_pcb, a_pcm, pri, cb, cm, m_pv, m_pcb, m_pcm, Wp, bp)


def kernel(adjacency, adjacency_pv, adjacency_vp, adjacency_pcb,
           adjacency_cbp, adjacency_cbv, adjacency_vcb, adjacency_pcm,
           adjacency_cmp, adjacency_cmv, adjacency_vcm, adjacency_cbcm,
           adjacency_cmcb, item_emb, pri_emb, cateBig_emb, cateMiddle_emb,
           mat_vp, mat_vcb, mat_vcm, mat_pv, mat_pcb, mat_pcm, mat_cbp,
           mat_cbv, mat_cbcm, mat_cmp, mat_cmv, mat_cmcb, W_gi, b_gi,
           W_gp, b_gp, W_gcb, b_gcb, W_gcm, b_gcm):
    b_gi2 = b_gi.reshape(1, 1)
    b_gp2 = b_gp.reshape(1, 1)
    b_gcb2 = b_gcb.reshape(1, 1)
    b_gcm2 = b_gcm.reshape(1, 1)

    it1 = _item_update(adjacency, adjacency_vp, adjacency_vcb, adjacency_vcm,
                       item_emb, pri_emb, cateBig_emb, cateMiddle_emb,
                       mat_vp, mat_vcb, mat_vcm, W_gi, b_gi2)
    pr1, cb1, cm1 = _pcc_update(
        adjacency_pv, adjacency_pcb, adjacency_pcm, adjacency_cbp,
        adjacency_cbv, adjacency_cbcm, adjacency_cmp, adjacency_cmv,
        adjacency_cmcb, item_emb, pri_emb, cateBig_emb, cateMiddle_emb,
        mat_pv, mat_pcb, mat_pcm, mat_cbp, mat_cbv, mat_cbcm, mat_cmp,
        mat_cmv, mat_cmcb, W_gp, b_gp2, W_gcb, b_gcb2, W_gcm, b_gcm2)

    it2, pr2 = _item_pr_update(
        adjacency, adjacency_vp, adjacency_vcb, adjacency_vcm,
        it1, pr1, cb1, cm1, mat_vp, mat_vcb, mat_vcm, W_gi, b_gi2,
        adjacency_pv, adjacency_pcb, adjacency_pcm,
        mat_pv, mat_pcb, mat_pcm, W_gp, b_gp2)
    return (it2, pr2)
